# baseline (speedup 1.0000x reference)
import dataclasses
import functools

import jax
import jax.numpy as jnp
from jax import lax
from jax.experimental import pallas as pl
from jax.experimental.pallas import tpu as pltpu
from jax.experimental.pallas import tpu_sc as plsc

N, E, IN, HID, C, OUT = 10000, 320000, 128, 128, 2, 128
H = HID * C

LANES = 16
NTILE = 16
EPT = E // NTILE
K = 80
NCHUNK = EPT // K
WB = 80
NWB = N // WB

NB = 10
BLK = N // NB

_PREC = jax.lax.Precision.HIGHEST


def _front_body(x_ref, w1t_ref, b1_ref, g0_ref, be0_ref, wct_ref, hn_ref, xc_ref):
    h = jnp.dot(x_ref[...], w1t_ref[...], precision=_PREC) + b1_ref[...]
    h = jnp.maximum(h, 0.0)
    m = jnp.mean(h, axis=-1, keepdims=True)
    v = jnp.mean((h - m) ** 2, axis=-1, keepdims=True)
    hn = (h - m) / jnp.sqrt(v + 1e-5) * g0_ref[...] + be0_ref[...]
    hn_ref[...] = hn
    xc_ref[...] = jnp.dot(hn, wct_ref[...], precision=_PREC)


def _dense_front(x, w1t, b1, g0, be0, wct):
    return pl.pallas_call(
        _front_body,
        grid=(NB,),
        in_specs=[
            pl.BlockSpec((BLK, IN), lambda i: (i, 0)),
            pl.BlockSpec((IN, H), lambda i: (0, 0)),
            pl.BlockSpec((1, H), lambda i: (0, 0)),
            pl.BlockSpec((1, H), lambda i: (0, 0)),
            pl.BlockSpec((1, H), lambda i: (0, 0)),
            pl.BlockSpec((H, HID), lambda i: (0, 0)),
        ],
        out_specs=[
            pl.BlockSpec((BLK, H), lambda i: (i, 0)),
            pl.BlockSpec((BLK, HID), lambda i: (i, 0)),
        ],
        out_shape=[
            jax.ShapeDtypeStruct((N, H), jnp.float32),
            jax.ShapeDtypeStruct((N, HID), jnp.float32),
        ],
    )(x, w1t, b1, g0, be0, wct)


def _back_body(agg_ref, hn_ref, g1_ref, be1_ref, w2t_ref, b2_ref, out_ref):
    a = jnp.concatenate([agg_ref[0], agg_ref[1]], axis=-1)
    h2 = jnp.maximum(a, 0.0)
    m = jnp.mean(h2, axis=-1, keepdims=True)
    v = jnp.mean((h2 - m) ** 2, axis=-1, keepdims=True)
    h2 = (h2 - m) / jnp.sqrt(v + 1e-5) * g1_ref[...] + be1_ref[...]
    h = 0.5 * h2 + 0.5 * hn_ref[...]
    out_ref[...] = jnp.dot(h, w2t_ref[...], precision=_PREC) + b2_ref[...]


def _dense_back(agg2, hn, g1, be1, w2t, b2):
    return pl.pallas_call(
        _back_body,
        grid=(NB,),
        in_specs=[
            pl.BlockSpec((2, BLK, HID), lambda i: (0, i, 0)),
            pl.BlockSpec((BLK, H), lambda i: (i, 0)),
            pl.BlockSpec((1, H), lambda i: (0, 0)),
            pl.BlockSpec((1, H), lambda i: (0, 0)),
            pl.BlockSpec((H, OUT), lambda i: (0, 0)),
            pl.BlockSpec((1, OUT), lambda i: (0, 0)),
        ],
        out_specs=pl.BlockSpec((BLK, OUT), lambda i: (i, 0)),
        out_shape=jax.ShapeDtypeStruct((N, OUT), jnp.float32),
    )(agg2, hn, g1, be1, w2t, b2)


def _edge_body(xc_hbm, row_hbm, col_hbm, wd_hbm, out_hbm,
               rowv, colv, abuf, bbuf, wbuf, wdv, aggsh, sem_a, sem_b):
    c = lax.axis_index("c")
    s = lax.axis_index("s")
    sign = (1 - 2 * c).astype(jnp.float32)
    lane = lax.iota(jnp.int32, LANES)

    @pl.loop(0, K)
    def _z(i):
        @pl.loop(0, HID, step=LANES)
        def _zz(j):
            bbuf[i, pl.ds(j, LANES)] = jnp.zeros((LANES,), jnp.float32)

    @pl.loop(0, (NWB + NTILE - 1) // NTILE)
    def _zc(j):
        cid = s + NTILE * j

        @pl.when(cid < NWB)
        def _():
            pltpu.sync_copy(bbuf, aggsh.at[pl.ds(cid * WB, WB)])

    pltpu.sync_copy(wd_hbm, wdv.at[pl.ds(0, HID)])
    plsc.subcore_barrier()

    @pl.loop(0, NCHUNK)
    def _chunk(g):
        off = s * EPT + g * K
        pltpu.sync_copy(row_hbm.at[pl.ds(off, K)], rowv)
        pltpu.sync_copy(col_hbm.at[pl.ds(off, K)], colv)
        cp_a = pltpu.async_copy(xc_hbm.at[rowv], abuf, sem_a)
        cp_b = pltpu.async_copy(xc_hbm.at[colv], bbuf, sem_b)
        cp_a.wait()
        cp_b.wait()

        @pl.loop(0, K, step=LANES)
        def _grp(e0):
            rows = rowv[pl.ds(e0, LANES)]
            cols = colv[pl.ds(e0, LANES)]
            eids = e0 + lane

            def fbody(f, acc):
                fv = jnp.full((LANES,), f, jnp.int32)
                av = plsc.load_gather(abuf, [eids, fv])
                bv = plsc.load_gather(bbuf, [eids, fv])
                t = jnp.maximum(0.5 * av + bv, 0.0)
                wf = wdv[pl.ds(f, LANES)][0]
                return acc + t * wf

            acc = lax.fori_loop(0, HID, fbody, jnp.zeros((LANES,), jnp.float32))
            d = sign * acc
            w = 1.0 / (1.0 + jnp.exp(-d))
            w = jnp.where(rows != cols, w, 0.0)
            wbuf[pl.ds(e0, LANES)] = w

        @pl.loop(0, K)
        def _scale(e):
            w = wbuf[pl.ds(e, LANES)][0]
            for k2 in range(HID // LANES):
                sl = pl.ds(k2 * LANES, LANES)
                bbuf[e, sl] = bbuf[e, sl] * w

        pltpu.sync_copy(bbuf, aggsh.at[rowv], add=True)

    plsc.subcore_barrier()

    @pl.loop(0, (NWB + NTILE - 1) // NTILE)
    def _out(j):
        cid = s + NTILE * j

        @pl.when(cid < NWB)
        def _():
            r0 = cid * WB
            pltpu.sync_copy(aggsh.at[pl.ds(r0, WB)],
                            out_hbm.at[c].at[pl.ds(r0, WB)])


def _edge_sc(xc, row, col, wd):
    mesh = plsc.VectorSubcoreMesh(core_axis_name="c", subcore_axis_name="s")
    cp = pltpu.CompilerParams()
    if "needs_layout_passes" in pltpu.CompilerParams.__dataclass_fields__:
        cp = dataclasses.replace(cp, needs_layout_passes=False)
    f = pl.kernel(
        _edge_body,
        out_type=jax.ShapeDtypeStruct((2, N, HID), jnp.float32),
        mesh=mesh,
        scratch_types=[
            pltpu.VMEM((K,), jnp.int32),
            pltpu.VMEM((K,), jnp.int32),
            pltpu.VMEM((K, HID), jnp.float32),
            pltpu.VMEM((K, HID), jnp.float32),
            pltpu.VMEM((K + LANES,), jnp.float32),
            pltpu.VMEM((HID + LANES,), jnp.float32),
            pltpu.VMEM_SHARED((N, HID), jnp.float32),
            pltpu.SemaphoreType.DMA,
            pltpu.SemaphoreType.DMA,
        ],
        compiler_params=cp,
    )
    return f(xc, row, col, wd)


def kernel(x, edge_index, W1, b1, g0, be0, Wconv, Watt, g1, be1, W2, b2):
    hn, xc = _dense_front(x, W1.T, b1.reshape(1, H), g0.reshape(1, H),
                          be0.reshape(1, H), Wconv.T)
    wd = Watt[0] - Watt[1]
    agg2 = _edge_sc(xc, edge_index[0], edge_index[1], wd)
    return _dense_back(agg2, hn, g1.reshape(1, H), be1.reshape(1, H),
                       W2.T, b2.reshape(1, OUT))


# speedup vs baseline: 4.5272x; 4.5272x over previous
import dataclasses
import functools

import jax
import jax.numpy as jnp
from jax import lax
from jax.experimental import pallas as pl
from jax.experimental.pallas import tpu as pltpu
from jax.experimental.pallas import tpu_sc as plsc

N, E, IN, HID, C, OUT = 10000, 320000, 128, 128, 2, 128
H = HID * C

LANES = 16
NTILE = 16
EPT = E // NTILE
K = 80
NCHUNK = EPT // K
WB = 80
NWB = N // WB

NB = 10
BLK = N // NB

_PREC = jax.lax.Precision.HIGHEST


def _front_body(x_ref, w1t_ref, b1_ref, g0_ref, be0_ref, wct_ref, hn_ref, xc_ref):
    h = jnp.dot(x_ref[...], w1t_ref[...], precision=_PREC) + b1_ref[...]
    h = jnp.maximum(h, 0.0)
    m = jnp.mean(h, axis=-1, keepdims=True)
    v = jnp.mean((h - m) ** 2, axis=-1, keepdims=True)
    hn = (h - m) / jnp.sqrt(v + 1e-5) * g0_ref[...] + be0_ref[...]
    hn_ref[...] = hn
    xc_ref[...] = jnp.dot(hn, wct_ref[...], precision=_PREC)


def _dense_front(x, w1t, b1, g0, be0, wct):
    return pl.pallas_call(
        _front_body,
        grid=(NB,),
        in_specs=[
            pl.BlockSpec((BLK, IN), lambda i: (i, 0)),
            pl.BlockSpec((IN, H), lambda i: (0, 0)),
            pl.BlockSpec((1, H), lambda i: (0, 0)),
            pl.BlockSpec((1, H), lambda i: (0, 0)),
            pl.BlockSpec((1, H), lambda i: (0, 0)),
            pl.BlockSpec((H, HID), lambda i: (0, 0)),
        ],
        out_specs=[
            pl.BlockSpec((BLK, H), lambda i: (i, 0)),
            pl.BlockSpec((BLK, HID), lambda i: (i, 0)),
        ],
        out_shape=[
            jax.ShapeDtypeStruct((N, H), jnp.float32),
            jax.ShapeDtypeStruct((N, HID), jnp.float32),
        ],
    )(x, w1t, b1, g0, be0, wct)


def _back_body(agg_ref, hn_ref, g1_ref, be1_ref, w2t_ref, b2_ref, out_ref):
    a = jnp.concatenate([agg_ref[0], agg_ref[1]], axis=-1)
    h2 = jnp.maximum(a, 0.0)
    m = jnp.mean(h2, axis=-1, keepdims=True)
    v = jnp.mean((h2 - m) ** 2, axis=-1, keepdims=True)
    h2 = (h2 - m) / jnp.sqrt(v + 1e-5) * g1_ref[...] + be1_ref[...]
    h = 0.5 * h2 + 0.5 * hn_ref[...]
    out_ref[...] = jnp.dot(h, w2t_ref[...], precision=_PREC) + b2_ref[...]


def _dense_back(agg2, hn, g1, be1, w2t, b2):
    return pl.pallas_call(
        _back_body,
        grid=(NB,),
        in_specs=[
            pl.BlockSpec((2, BLK, HID), lambda i: (0, i, 0)),
            pl.BlockSpec((BLK, H), lambda i: (i, 0)),
            pl.BlockSpec((1, H), lambda i: (0, 0)),
            pl.BlockSpec((1, H), lambda i: (0, 0)),
            pl.BlockSpec((H, OUT), lambda i: (0, 0)),
            pl.BlockSpec((1, OUT), lambda i: (0, 0)),
        ],
        out_specs=pl.BlockSpec((BLK, OUT), lambda i: (i, 0)),
        out_shape=jax.ShapeDtypeStruct((N, OUT), jnp.float32),
    )(agg2, hn, g1, be1, w2t, b2)


def _edge_body(xc_hbm, ei_hbm, wd_hbm, out_hbm,
               idx0, idx1, abuf0, bbuf0, abuf1, bbuf1, wdv, aggsh,
               sem_a0, sem_b0, sem_a1, sem_b1):
    c = lax.axis_index("c")
    s = lax.axis_index("s")
    sign = (1 - 2 * c).astype(jnp.float32)
    lane = lax.iota(jnp.int32, LANES)
    idxs = (idx0, idx1)
    abufs, bbufs = (abuf0, abuf1), (bbuf0, bbuf1)
    sems_a, sems_b = (sem_a0, sem_a1), (sem_b0, sem_b1)
    NK = HID // LANES

    @pl.loop(0, K)
    def _z(i):
        @pl.loop(0, HID, step=LANES)
        def _zz(j):
            bbuf0[i, pl.ds(j, LANES)] = jnp.zeros((LANES,), jnp.float32)

    @pl.loop(0, (NWB + NTILE - 1) // NTILE)
    def _zc(j):
        cid = s + NTILE * j

        @pl.when(cid < NWB)
        def _():
            pltpu.sync_copy(bbuf0, aggsh.at[pl.ds(cid * WB, WB)])

    pltpu.sync_copy(wd_hbm, wdv)
    wdk = [wdv[pl.ds(k * LANES, LANES)] for k in range(NK)]
    plsc.subcore_barrier()

    for p in range(2):
        pltpu.sync_copy(ei_hbm.at[s].at[p], idxs[p])
        pltpu.async_copy(xc_hbm.at[idxs[p].at[0]], abufs[p], sems_a[p])
        pltpu.async_copy(xc_hbm.at[idxs[p].at[1]], bbufs[p], sems_b[p])

    @pl.loop(0, NCHUNK // 2)
    def _pair(t):
        for p in range(2):
            g = 2 * t + p
            ab, bb = abufs[p], bbufs[p]
            ix = idxs[p]
            pltpu.make_async_copy(xc_hbm.at[ix.at[0]], ab, sems_a[p]).wait()
            pltpu.make_async_copy(xc_hbm.at[ix.at[1]], bb, sems_b[p]).wait()

            @pl.loop(0, K, step=LANES)
            def _grp(e0):
                rows = ix[0, pl.ds(e0, LANES)]
                cols = ix[1, pl.ds(e0, LANES)]
                dvec = jnp.zeros((LANES,), jnp.float32)
                for i in range(LANES):
                    e = e0 + i
                    acc = None
                    for k in range(NK):
                        sl = pl.ds(k * LANES, LANES)
                        va = ab[e, sl]
                        vb = bb[e, sl]
                        t_ = jnp.maximum(0.5 * va + vb, 0.0)
                        acc = t_ * wdk[k] if acc is None else acc + t_ * wdk[k]
                    d = jnp.sum(acc)
                    dvec = jnp.where(lane == i, d, dvec)
                w = 1.0 / (1.0 + jnp.exp(-sign * dvec))
                w = jnp.where(rows != cols, w, 0.0)
                for i in range(LANES):
                    e = e0 + i
                    wsc = w[i]
                    for k in range(NK):
                        sl = pl.ds(k * LANES, LANES)
                        bb[e, sl] = bb[e, sl] * wsc

            pltpu.sync_copy(bb, aggsh.at[ix.at[0]], add=True)

            @pl.when(g + 2 < NCHUNK)
            def _pf():
                pltpu.sync_copy(ei_hbm.at[s].at[g + 2], ix)
                pltpu.async_copy(xc_hbm.at[ix.at[0]], ab, sems_a[p])
                pltpu.async_copy(xc_hbm.at[ix.at[1]], bb, sems_b[p])

    plsc.subcore_barrier()

    @pl.loop(0, (NWB + NTILE - 1) // NTILE)
    def _out(j):
        cid = s + NTILE * j

        @pl.when(cid < NWB)
        def _():
            r0 = cid * WB
            pltpu.sync_copy(aggsh.at[pl.ds(r0, WB)],
                            out_hbm.at[c].at[pl.ds(r0, WB)])


def _edge_sc(xc, row, col, wd):
    mesh = plsc.VectorSubcoreMesh(core_axis_name="c", subcore_axis_name="s")
    cp = pltpu.CompilerParams()
    if "needs_layout_passes" in pltpu.CompilerParams.__dataclass_fields__:
        cp = dataclasses.replace(cp, needs_layout_passes=False)
    f = pl.kernel(
        _edge_body,
        out_type=jax.ShapeDtypeStruct((2, N, HID), jnp.float32),
        mesh=mesh,
        scratch_types=[
            pltpu.VMEM((2, K), jnp.int32),
            pltpu.VMEM((2, K), jnp.int32),
            pltpu.VMEM((K, HID), jnp.float32),
            pltpu.VMEM((K, HID), jnp.float32),
            pltpu.VMEM((K, HID), jnp.float32),
            pltpu.VMEM((K, HID), jnp.float32),
            pltpu.VMEM((HID,), jnp.float32),
            pltpu.VMEM_SHARED((N, HID), jnp.float32),
            pltpu.SemaphoreType.DMA,
            pltpu.SemaphoreType.DMA,
            pltpu.SemaphoreType.DMA,
            pltpu.SemaphoreType.DMA,
        ],
        compiler_params=cp,
    )
    ei = jnp.stack([row.reshape(NTILE, NCHUNK, K),
                    col.reshape(NTILE, NCHUNK, K)], axis=2)
    return f(xc, ei, wd)


def kernel(x, edge_index, W1, b1, g0, be0, Wconv, Watt, g1, be1, W2, b2):
    hn, xc = _dense_front(x, W1.T, b1.reshape(1, H), g0.reshape(1, H),
                          be0.reshape(1, H), Wconv.T)
    wd = Watt[0] - Watt[1]
    agg2 = _edge_sc(xc, edge_index[0], edge_index[1], wd)
    return _dense_back(agg2, hn, g1.reshape(1, H), be1.reshape(1, H),
                       W2.T, b2.reshape(1, OUT))
